# baseline (speedup 1.0000x reference)
import jax
import jax.numpy as jnp
from jax import lax
from jax.experimental import pallas as pl
from jax.experimental.pallas import tpu as pltpu
from jax.experimental.pallas import tpu_sc as plsc

_VOCAB = 1000000
_EMBED = 64
_BATCH = 4096
_SEQ = 200
_HALF = _EMBED // 2

_NC = 2
_NS = 16
_NW = _NC * _NS
_BPW = _BATCH // _NW
_NBUF = 2
_KMAX = _SEQ // _NBUF


def _sc_body(x, sincos, table, out5,
             idx_v, sc_v, gidx,
             r0, r1, t0, t1,
             si0, si1, so0, so1):
    rows = [r0, r1]
    tbs = [t0, t1]
    sin_ = [si0, si1]
    sout = [so0, so1]
    wid = lax.axis_index("s") * _NC + lax.axis_index("c")

    pltpu.sync_copy(x.at[pl.ds(wid * _BPW, _BPW)], idx_v)
    pltpu.sync_copy(sincos, sc_v)

    iota = lax.iota(jnp.int32, 16)
    rowidx = [iota + 16 * i for i in range(4)]

    def block(k, carry):
        hin = []
        for b in range(_NBUF):
            t = _NBUF * k + b
            tvec = jnp.zeros((16,), jnp.int32) + t
            for m in range(_BPW // 16):
                vals = plsc.load_gather(idx_v, [iota + 16 * m, tvec])
                gidx[b, pl.ds(16 * m, 16)] = vals
            hin.append(pltpu.async_copy(
                table.at[gidx.at[b]], rows[b], sin_[b]))

        hout = []
        for b in range(_NBUF):
            t = _NBUF * k + b
            hin[b].wait()
            rb = rows[b]
            tbb = tbs[b]
            c0 = sc_v[t, 0:16]
            c1 = sc_v[t, 16:32]
            s0 = sc_v[t, 32:48]
            s1 = sc_v[t, 48:64]

            @plsc.parallel_loop(0, _BPW, unroll=2)
            def _(j):
                e0 = rb[j, 0:16]
                e1 = rb[j, 16:32]
                o0 = rb[j, 32:48]
                o1 = rb[j, 48:64]
                jvec = jnp.zeros((16,), jnp.int32) + j
                plsc.store_scatter(tbb, [rowidx[0], jvec], e0 * c0 - o0 * s0)
                plsc.store_scatter(tbb, [rowidx[1], jvec], e1 * c1 - o1 * s1)
                plsc.store_scatter(tbb, [rowidx[2], jvec], e0 * s0 + o0 * c0)
                plsc.store_scatter(tbb, [rowidx[3], jvec], e1 * s1 + o1 * c1)

            for cb in range(_EMBED // 8):
                hout.append(pltpu.async_copy(
                    tbb.at[pl.ds(cb * 8, 8)], out5.at[t, cb, wid], sout[b]))
        for h in hout:
            h.wait()
        return carry

    lax.fori_loop(0, _KMAX, block, 0)


@jax.jit
def _sc_call(x, sincos, table):
    mesh = plsc.VectorSubcoreMesh(core_axis_name="c", subcore_axis_name="s")
    f = pl.kernel(
        _sc_body,
        mesh=mesh,
        compiler_params=pltpu.CompilerParams(
            use_tc_tiling_on_sc=False, needs_layout_passes=False),
        out_type=jax.ShapeDtypeStruct((_SEQ, 8, _NW, 8, 128), jnp.float32),
        scratch_types=[
            pltpu.VMEM((_BPW, _SEQ), jnp.int32),
            pltpu.VMEM((_SEQ, _EMBED), jnp.float32),
            pltpu.VMEM((_NBUF, _BPW), jnp.int32),
        ] + [pltpu.VMEM((_BPW, 128), jnp.float32)] * _NBUF
          + [pltpu.VMEM((_EMBED, 128), jnp.float32)] * _NBUF
          + [pltpu.SemaphoreType.DMA] * (2 * _NBUF),
    )
    return f(x, sincos, table)


def kernel(x, table):
    if x.ndim == 1:
        x = x[None, :]
    x = x.astype(jnp.int32)
    table128 = jnp.pad(table, ((0, 0), (0, 128 - _EMBED)))
    freqs = 1.0 / (10000.0 ** (jnp.arange(_HALF, dtype=jnp.float32) / _EMBED))
    ang = jnp.arange(_SEQ, dtype=jnp.float32)[:, None] * freqs[None, :]
    sincos = jnp.concatenate([jnp.cos(ang), jnp.sin(ang)], axis=-1)
    out5 = _sc_call(x, sincos, table128)
    return out5.transpose(2, 4, 0, 1, 3).reshape(_BATCH, _SEQ, _EMBED)


# speedup vs baseline: 1.0712x; 1.0712x over previous
import jax
import jax.numpy as jnp
from jax import lax
from jax.experimental import pallas as pl
from jax.experimental.pallas import tpu as pltpu
from jax.experimental.pallas import tpu_sc as plsc

_VOCAB = 1000000
_EMBED = 64
_BATCH = 4096
_SEQ = 200
_HALF = _EMBED // 2

_NC = 2
_NS = 16
_NW = _NC * _NS
_BPW = _BATCH // _NW
_NBUF = 2
_KMAX = _SEQ // _NBUF


def _sc_body(x, sincos, table, out5,
             idx_v, sc_v, gidx,
             r0, r1, t0, t1,
             si0, si1, so0, so1):
    rows = [r0, r1]
    tbs = [t0, t1]
    sin_ = [si0, si1]
    sout = [so0, so1]
    wid = lax.axis_index("s") * _NC + lax.axis_index("c")

    pltpu.sync_copy(x.at[pl.ds(wid * _BPW, _BPW)], idx_v)
    pltpu.sync_copy(sincos, sc_v)

    iota = lax.iota(jnp.int32, 16)
    cbi = [(iota + 16 * i) >> 3 for i in range(4)]
    csi = [(iota + 16 * i) & 7 for i in range(4)]
    ones = jnp.zeros((16,), jnp.int32) + 1

    def block(k, carry):
        hin = []
        for b in range(_NBUF):
            t = _NBUF * k + b
            tvec = jnp.zeros((16,), jnp.int32) + t
            for m in range(_BPW // 16):
                vals = plsc.load_gather(idx_v, [iota + 16 * m, tvec])
                gidx[b, pl.ds(16 * m, 16)] = vals
            hin.append(pltpu.async_copy(
                table.at[gidx.at[b]], rows[b], sin_[b]))

        hout = []
        for b in range(_NBUF):
            t = _NBUF * k + b
            hin[b].wait()
            rb = rows[b]
            tbb = tbs[b]
            c0 = sc_v[t, 0:16]
            c1 = sc_v[t, 16:32]
            s0 = sc_v[t, 32:48]
            s1 = sc_v[t, 48:64]

            @plsc.parallel_loop(0, _BPW, unroll=4, carry=jnp.zeros((16,), jnp.int32))
            def _(j, jvec):
                e0 = rb[j, 0:16]
                e1 = rb[j, 16:32]
                o0 = rb[j, 32:48]
                o1 = rb[j, 48:64]
                plsc.store_scatter(tbb, [cbi[0], csi[0], jvec], e0 * c0 - o0 * s0)
                plsc.store_scatter(tbb, [cbi[1], csi[1], jvec], e1 * c1 - o1 * s1)
                plsc.store_scatter(tbb, [cbi[2], csi[2], jvec], e0 * s0 + o0 * c0)
                plsc.store_scatter(tbb, [cbi[3], csi[3], jvec], e1 * s1 + o1 * c1)
                return jvec + ones

            hout.append(pltpu.async_copy(
                tbb, out5.at[t, pl.ds(0, 8), wid], sout[b]))
        for h in hout:
            h.wait()
        return carry

    lax.fori_loop(0, _KMAX, block, 0)


@jax.jit
def _sc_call(x, sincos, table):
    mesh = plsc.VectorSubcoreMesh(core_axis_name="c", subcore_axis_name="s")
    f = pl.kernel(
        _sc_body,
        mesh=mesh,
        compiler_params=pltpu.CompilerParams(
            use_tc_tiling_on_sc=False, needs_layout_passes=False),
        out_type=jax.ShapeDtypeStruct((_SEQ, 8, _NW, 8, 128), jnp.float32),
        scratch_types=[
            pltpu.VMEM((_BPW, _SEQ), jnp.int32),
            pltpu.VMEM((_SEQ, _EMBED), jnp.float32),
            pltpu.VMEM((_NBUF, _BPW), jnp.int32),
        ] + [pltpu.VMEM((_BPW, 128), jnp.float32)] * _NBUF
          + [pltpu.VMEM((8, 8, 128), jnp.float32)] * _NBUF
          + [pltpu.SemaphoreType.DMA] * (2 * _NBUF),
    )
    return f(x, sincos, table)


def kernel(x, table):
    if x.ndim == 1:
        x = x[None, :]
    x = x.astype(jnp.int32)
    table128 = jnp.pad(table, ((0, 0), (0, 128 - _EMBED)))
    freqs = 1.0 / (10000.0 ** (jnp.arange(_HALF, dtype=jnp.float32) / _EMBED))
    ang = jnp.arange(_SEQ, dtype=jnp.float32)[:, None] * freqs[None, :]
    sincos = jnp.concatenate([jnp.cos(ang), jnp.sin(ang)], axis=-1)
    out5 = _sc_call(x, sincos, table128)
    return out5.transpose(2, 4, 0, 1, 3).reshape(_BATCH, _SEQ, _EMBED)


# speedup vs baseline: 1.2893x; 1.2036x over previous
import jax
import jax.numpy as jnp
from jax import lax
from jax.experimental import pallas as pl
from jax.experimental.pallas import tpu as pltpu
from jax.experimental.pallas import tpu_sc as plsc

_VOCAB = 1000000
_EMBED = 64
_BATCH = 4096
_SEQ = 200
_HALF = _EMBED // 2

_NC = 2
_NS = 16
_NW = _NC * _NS

_BPW = _BATCH // _NW
_RPC = 2
_SPLITS = ((0, 104), (104, 96))
_NBUF = 4
_ROWS_PER_IT = _RPC * _NBUF
_KMAX = _BPW // _ROWS_PER_IT


def _sc_body(x, sincos, table, out,
             idx_v, sc_v,
             r0, r1, r2, r3,
             si0, si1, si2, si3,
             so0, so1, so2, so3):
    rows = [r0, r1, r2, r3]
    sin_ = [si0, si1, si2, si3]
    sout = [so0, so1, so2, so3]
    wid = lax.axis_index("s") * _NC + lax.axis_index("c")

    pltpu.sync_copy(sincos, sc_v)

    def block(k, carry):
        brow0 = wid * _BPW + _ROWS_PER_IT * k
        pltpu.sync_copy(x.at[pl.ds(brow0, _ROWS_PER_IT)], idx_v)

        hin = []
        for b in range(_NBUF):
            for s in range(_RPC):
                for (off, width) in _SPLITS:
                    hin.append(pltpu.async_copy(
                        table.at[idx_v.at[_RPC * b + s, pl.ds(off, width)]],
                        rows[b].at[s, pl.ds(off, width)],
                        sin_[b],
                    ))

        npc = _RPC * len(_SPLITS)
        hout = []
        for b in range(_NBUF):
            for h in hin[npc * b:npc * (b + 1)]:
                h.wait()

            rb = rows[b]

            @plsc.parallel_loop(0, _SEQ, unroll=2)
            def _(t):
                c0 = sc_v[t, 0:16]
                c1 = sc_v[t, 16:32]
                s0 = sc_v[t, 32:48]
                s1 = sc_v[t, 48:64]
                for s in range(_RPC):
                    e0 = rb[s, t, 0:16]
                    e1 = rb[s, t, 16:32]
                    o0 = rb[s, t, 32:48]
                    o1 = rb[s, t, 48:64]
                    rb[s, t, 0:16] = e0 * c0 - o0 * s0
                    rb[s, t, 16:32] = e1 * c1 - o1 * s1
                    rb[s, t, 32:48] = e0 * s0 + o0 * c0
                    rb[s, t, 48:64] = e1 * s1 + o1 * c1

            hout.append(pltpu.async_copy(
                rb, out.at[pl.ds(brow0 + _RPC * b, _RPC)], sout[b]))
        for h in hout:
            h.wait()
        return carry

    lax.fori_loop(0, _KMAX, block, 0)


@jax.jit
def _sc_call(x, sincos, table):
    mesh = plsc.VectorSubcoreMesh(core_axis_name="c", subcore_axis_name="s")
    f = pl.kernel(
        _sc_body,
        mesh=mesh,
        compiler_params=pltpu.CompilerParams(use_tc_tiling_on_sc=False),
        out_type=jax.ShapeDtypeStruct((_BATCH, _SEQ, _EMBED), jnp.float32),
        scratch_types=[
            pltpu.VMEM((_ROWS_PER_IT, _SEQ), jnp.int32),
            pltpu.VMEM((_SEQ, _EMBED), jnp.float32),
        ] + [pltpu.VMEM((_RPC, _SEQ, _EMBED), jnp.float32)] * _NBUF
          + [pltpu.SemaphoreType.DMA] * (2 * _NBUF),
    )
    return f(x, sincos, table)


def kernel(x, table):
    if x.ndim == 1:
        x = x[None, :]
    x = x.astype(jnp.int32)
    freqs = 1.0 / (10000.0 ** (jnp.arange(_HALF, dtype=jnp.float32) / _EMBED))
    ang = jnp.arange(_SEQ, dtype=jnp.float32)[:, None] * freqs[None, :]
    sincos = jnp.concatenate([jnp.cos(ang), jnp.sin(ang)], axis=-1)
    return _sc_call(x, sincos, table)
